# baseline (speedup 1.0000x reference)
import functools

import jax
import jax.numpy as jnp
from jax import lax
from jax.experimental import pallas as pl
from jax.experimental.pallas import tpu as pltpu
from jax.experimental.pallas import tpu_sc as plsc

N = 16384
D = 256
K = 8192
BN = 256
BK = 1024
NI = N // BN
NJ = K // BK


SEG = 256
ROUND_COLS = (2816, 5632)


def _argmin_body(x_ref, e_ref, idx_ref, counts_ref):
    i = pl.program_id(0)
    x_blk = x_ref[...]
    e_blk = e_ref[...]

    x2 = jnp.sum(x_blk * x_blk, axis=1, keepdims=True)

    best = bidx = None

    def seg_min(dist, lo, hi, base):
        seg = dist[:, lo:hi]
        m = jnp.min(seg, axis=1, keepdims=True)
        a = (jnp.argmin(seg, axis=1).astype(jnp.int32).reshape(BN, 1)
             + (base + lo))
        return m, a

    for c0 in range(0, K, BK):
        e_chunk = e_blk[:, c0:c0 + BK]
        xe = jnp.dot(x_blk, e_chunk)
        e2 = jnp.sum(e_chunk * e_chunk, axis=0, keepdims=True)
        dist = (x2 - 2.0 * xe) + e2
        cuts = [rc - c0 for rc in ROUND_COLS if c0 < rc < c0 + BK]
        edges = [0] + cuts + [BK]
        for lo, hi in zip(edges[:-1], edges[1:]):
            m, a = seg_min(dist, lo, hi, c0)
            if best is None:
                best, bidx = m, a
            else:
                upd = m < best
                best = jnp.where(upd, m, best)
                bidx = jnp.where(upd, a, bidx)
            if c0 + hi in ROUND_COLS:
                best = best.astype(jnp.bfloat16).astype(jnp.float32)

    idx_ref[...] = bidx
    onehot = (bidx == lax.broadcasted_iota(jnp.int32, (1, K), 1)
              ).astype(jnp.float32)
    c = jnp.sum(onehot, axis=0, keepdims=True)

    @pl.when(i == 0)
    def _():
        counts_ref[...] = c

    @pl.when(i > 0)
    def _():
        counts_ref[...] = counts_ref[...] + c


def _argmin_call(x, emb):
    return pl.pallas_call(
        _argmin_body,
        grid=(NI,),
        in_specs=[
            pl.BlockSpec((BN, D), lambda i: (i, 0)),
            pl.BlockSpec((D, K), lambda i: (0, 0)),
        ],
        out_specs=[
            pl.BlockSpec((BN, 1), lambda i: (i, 0)),
            pl.BlockSpec((1, K), lambda i: (0, 0)),
        ],
        out_shape=[
            jax.ShapeDtypeStruct((N, 1), jnp.int32),
            jax.ShapeDtypeStruct((1, K), jnp.float32),
        ],
    )(x, emb)


SC_CHUNK = 256
SC_WORKERS = 32
PER_W = N // SC_WORKERS


def _gather_call(emb_t, idx_flat):
    mesh = plsc.VectorSubcoreMesh(core_axis_name="c", subcore_axis_name="s")

    @functools.partial(
        pl.kernel,
        mesh=mesh,
        out_type=jax.ShapeDtypeStruct((N, D), jnp.float32),
        scratch_types=[
            pltpu.VMEM((SC_CHUNK,), jnp.int32),
            pltpu.VMEM((SC_CHUNK, D), jnp.float32),
            pltpu.SemaphoreType.DMA,
        ],
    )
    def gather_k(table_hbm, idx_hbm, out_hbm, idx_v, rows_v, sem):
        wid = lax.axis_index("s") * 2 + lax.axis_index("c")
        base = wid * PER_W
        for c in range(0, PER_W, SC_CHUNK):
            pltpu.sync_copy(idx_hbm.at[pl.ds(base + c, SC_CHUNK)], idx_v)
            pltpu.async_copy(table_hbm.at[idx_v], rows_v, sem).wait()
            pltpu.sync_copy(rows_v, out_hbm.at[pl.ds(base + c, SC_CHUNK)])

    return gather_k(emb_t, idx_flat)


BL = 1024
NL = N // BL


def _loss_body(q_ref, x_ref, counts_ref, loss_ref, perp_ref, acc_ref):
    i = pl.program_id(0)
    d = q_ref[...] - x_ref[...]
    s = jnp.sum(d * d)

    @pl.when(i == 0)
    def _():
        acc_ref[0, 0] = s

    @pl.when(i > 0)
    def _():
        acc_ref[0, 0] = acc_ref[0, 0] + s

    @pl.when(i == NL - 1)
    def _():
        loss_ref[...] = jnp.reshape(2.0 * (acc_ref[0, 0] / (N * D)), (1, 1))
        p = counts_ref[...] / N
        perp_ref[...] = jnp.reshape(
            jnp.exp(-jnp.sum(p * jnp.log(p + 1e-10))), (1, 1))


def _loss_call(q, x, counts):
    return pl.pallas_call(
        _loss_body,
        grid=(NL,),
        in_specs=[
            pl.BlockSpec((BL, D), lambda i: (i, 0)),
            pl.BlockSpec((BL, D), lambda i: (i, 0)),
            pl.BlockSpec((1, K), lambda i: (0, 0)),
        ],
        out_specs=[
            pl.BlockSpec((1, 1), lambda i: (0, 0)),
            pl.BlockSpec((1, 1), lambda i: (0, 0)),
        ],
        out_shape=[
            jax.ShapeDtypeStruct((1, 1), jnp.float32),
            jax.ShapeDtypeStruct((1, 1), jnp.float32),
        ],
        scratch_shapes=[pltpu.SMEM((1, 1), jnp.float32)],
    )(q, x, counts)


def kernel(inpt, emb_mtrx):
    x = inpt.reshape(N, D)
    idx, counts = _argmin_call(x, emb_mtrx)
    emb_t = emb_mtrx.T
    q = _gather_call(emb_t, idx.reshape(N))
    loss, perp = _loss_call(q, x, counts)
    return (q.reshape(inpt.shape), loss.reshape(()), perp.reshape(()))


# speedup vs baseline: 1.8752x; 1.8752x over previous
import functools

import jax
import jax.numpy as jnp
from jax import lax
from jax.experimental import pallas as pl
from jax.experimental.pallas import tpu as pltpu
from jax.experimental.pallas import tpu_sc as plsc

N = 16384
D = 256
K = 8192
BN = 256
BK = 1024
NI = N // BN
NJ = K // BK


SEG = 256
ROUND_COLS = (2816, 5632)


def _argmin_body(x_ref, e_ref, x2_ref, e2_ref, idx_ref, counts_ref):
    i = pl.program_id(0)
    x_blk = x_ref[...]
    e_blk = e_ref[...]

    x2 = x2_ref[...]

    xe = jnp.dot(x_blk, e_blk)
    dist = (x2 - 2.0 * xe) + e2_ref[...]

    def seg_min(lo, hi):
        seg = dist[:, lo:hi]
        m = jnp.min(seg, axis=1, keepdims=True)
        ii = lax.broadcasted_iota(jnp.int32, (BN, hi - lo), 1)
        a = jnp.min(jnp.where(seg == m, ii, K), axis=1, keepdims=True) + lo
        return m, a

    best, bidx = seg_min(0, ROUND_COLS[0])
    for lo, hi in ((ROUND_COLS[0], ROUND_COLS[1]), (ROUND_COLS[1], K)):
        best = best.astype(jnp.bfloat16).astype(jnp.float32)
        m, a = seg_min(lo, hi)
        upd = m < best
        best = jnp.where(upd, m, best)
        bidx = jnp.where(upd, a, bidx)

    idx_ref[...] = bidx
    onehot = (bidx == lax.broadcasted_iota(jnp.int32, (1, K), 1)
              ).astype(jnp.float32)
    c = jnp.sum(onehot, axis=0, keepdims=True)

    @pl.when(i == 0)
    def _():
        counts_ref[...] = c

    @pl.when(i > 0)
    def _():
        counts_ref[...] = counts_ref[...] + c


def _argmin_call(x, emb, x2, e2):
    return pl.pallas_call(
        _argmin_body,
        grid=(NI,),
        in_specs=[
            pl.BlockSpec((BN, D), lambda i: (i, 0)),
            pl.BlockSpec((D, K), lambda i: (0, 0)),
            pl.BlockSpec((BN, 1), lambda i: (i, 0)),
            pl.BlockSpec((1, K), lambda i: (0, 0)),
        ],
        out_specs=[
            pl.BlockSpec((BN, 1), lambda i: (i, 0)),
            pl.BlockSpec((1, K), lambda i: (0, 0)),
        ],
        out_shape=[
            jax.ShapeDtypeStruct((N, 1), jnp.int32),
            jax.ShapeDtypeStruct((1, K), jnp.float32),
        ],
    )(x, emb, x2, e2)


SC_CHUNK = 256
SC_WORKERS = 32
PER_W = N // SC_WORKERS


def _gather_call(emb_t, idx_flat):
    mesh = plsc.VectorSubcoreMesh(core_axis_name="c", subcore_axis_name="s")

    @functools.partial(
        pl.kernel,
        mesh=mesh,
        out_type=jax.ShapeDtypeStruct((N, D), jnp.float32),
        scratch_types=[
            pltpu.VMEM((SC_CHUNK,), jnp.int32),
            pltpu.VMEM((SC_CHUNK, D), jnp.float32),
            pltpu.SemaphoreType.DMA,
        ],
    )
    def gather_k(table_hbm, idx_hbm, out_hbm, idx_v, rows_v, sem):
        wid = lax.axis_index("s") * 2 + lax.axis_index("c")
        base = wid * PER_W
        for c in range(0, PER_W, SC_CHUNK):
            pltpu.sync_copy(idx_hbm.at[pl.ds(base + c, SC_CHUNK)], idx_v)
            pltpu.async_copy(table_hbm.at[idx_v], rows_v, sem).wait()
            pltpu.sync_copy(rows_v, out_hbm.at[pl.ds(base + c, SC_CHUNK)])

    return gather_k(emb_t, idx_flat)


BL = 1024
NL = N // BL


def _loss_body(q_ref, x_ref, counts_ref, loss_ref, perp_ref, acc_ref):
    i = pl.program_id(0)
    d = q_ref[...] - x_ref[...]
    s = jnp.sum(d * d)

    @pl.when(i == 0)
    def _():
        acc_ref[0, 0] = s

    @pl.when(i > 0)
    def _():
        acc_ref[0, 0] = acc_ref[0, 0] + s

    @pl.when(i == NL - 1)
    def _():
        loss_ref[...] = jnp.reshape(2.0 * (acc_ref[0, 0] / (N * D)), (1, 1))
        p = counts_ref[...] / N
        perp_ref[...] = jnp.reshape(
            jnp.exp(-jnp.sum(p * jnp.log(p + 1e-10))), (1, 1))


def _loss_call(q, x, counts):
    return pl.pallas_call(
        _loss_body,
        grid=(NL,),
        in_specs=[
            pl.BlockSpec((BL, D), lambda i: (i, 0)),
            pl.BlockSpec((BL, D), lambda i: (i, 0)),
            pl.BlockSpec((1, K), lambda i: (0, 0)),
        ],
        out_specs=[
            pl.BlockSpec((1, 1), lambda i: (0, 0)),
            pl.BlockSpec((1, 1), lambda i: (0, 0)),
        ],
        out_shape=[
            jax.ShapeDtypeStruct((1, 1), jnp.float32),
            jax.ShapeDtypeStruct((1, 1), jnp.float32),
        ],
        scratch_shapes=[pltpu.SMEM((1, 1), jnp.float32)],
    )(q, x, counts)


def kernel(inpt, emb_mtrx):
    x = inpt.reshape(N, D)
    x2 = jnp.sum(x ** 2, axis=1, keepdims=True)
    e2 = jnp.sum(emb_mtrx ** 2, axis=0, keepdims=True)
    idx, counts = _argmin_call(x, emb_mtrx, x2, e2)
    emb_t = emb_mtrx.T
    q = _gather_call(emb_t, idx.reshape(N))
    loss, perp = _loss_call(q, x, counts)
    return (q.reshape(inpt.shape), loss.reshape(()), perp.reshape(()))
